# Initial kernel scaffold; baseline (speedup 1.0000x reference)
import jax
import jax.numpy as jnp
from jax.experimental import pallas as pl


def kernel(feat0, feat1, feat2, feat3, rois):
    raise NotImplementedError("write your pallas kernel here")


# speedup vs baseline: 26.1439x; 26.1439x over previous
import functools

import jax
import jax.numpy as jnp
from jax import lax
from jax.experimental import pallas as pl
from jax.experimental.pallas import tpu as pltpu
from jax.experimental.pallas import tpu_sc as plsc

_OUT = 7
_SN = 2
_S14 = _OUT * _SN
_FINEST = 56.0
_SIZES = (256, 128, 64, 32)
_INV_STRIDES = (0.25, 0.125, 0.0625, 0.03125)
_NC = 2
_NS = 16
_NW = _NC * _NS
_CH = 8
_TERMS = 16
_PAIRS = _CH * _TERMS
_LANES = 16


def _axis_terms(lo, hi, size_i, size_f):
    K = lo.shape[0]
    roi = jnp.maximum(hi - lo, 1.0)
    binsz = roi / _OUT
    g = (jnp.arange(_S14, dtype=jnp.float32) + 0.5) / _SN
    coord = lo[:, None] + g[None, :] * binsz[:, None]
    limf = size_f[:, None]
    valid = jnp.logical_and(coord >= -1.0, coord <= limf)
    c = jnp.maximum(coord, 0.0)
    low0 = jnp.floor(c).astype(jnp.int32)
    cond = low0 >= (size_i[:, None] - 1)
    low = jnp.where(cond, size_i[:, None] - 1, low0)
    high = jnp.where(cond, size_i[:, None] - 1, low0 + 1)
    c = jnp.where(cond, limf - 1.0, c)
    l = c - low.astype(jnp.float32)
    wl = jnp.where(valid, 1.0 - l, 0.0)
    wh = jnp.where(valid, l, 0.0)
    pos = jnp.stack([low, high], axis=2).reshape(K, 2 * _S14)
    wt = jnp.stack([wl, wh], axis=2).reshape(K, 2 * _S14)
    return pos, wt


def _indices_weights(rois, bases):
    K = rois.shape[0]
    b = rois[:, 0].astype(jnp.int32)
    x1, y1, x2, y2 = rois[:, 1], rois[:, 2], rois[:, 3], rois[:, 4]
    scale = jnp.sqrt((x2 - x1 + 1.0) * (y2 - y1 + 1.0))
    lvl = jnp.clip(jnp.floor(jnp.log2(scale / _FINEST + 1e-6)), 0, 3).astype(jnp.int32)
    size = jnp.asarray(_SIZES, jnp.int32)[lvl]
    inv = jnp.asarray(_INV_STRIDES, jnp.float32)[lvl]
    base = jnp.asarray(bases, jnp.int32)[lvl] + b * size * size
    limf = size.astype(jnp.float32)
    ypos, yw = _axis_terms(y1 * inv, y2 * inv, size, limf)
    xpos, xw = _axis_terms(x1 * inv, x2 * inv, size, limf)
    idx = (base[:, None, None, None, None]
           + ypos.reshape(K, _OUT, 1, 4, 1) * size[:, None, None, None, None]
           + xpos.reshape(K, 1, _OUT, 1, 4))
    w = yw.reshape(K, _OUT, 1, 4, 1) * xw.reshape(K, 1, _OUT, 1, 4) * (1.0 / (_SN * _SN))
    return idx.reshape(K * _OUT * _OUT, _TERMS), w.reshape(K * _OUT * _OUT, _TERMS)


def _sc_body(table, idxt, wgtt, out, idxv, wgtv, gbuf, outv, sem):
    wid = lax.axis_index("s") * _NC + lax.axis_index("c")
    nch = idxt.shape[1]
    rows_per_tile = nch * _CH
    pltpu.sync_copy(idxt.at[wid], idxv)
    pltpu.sync_copy(wgtt.at[wid], wgtv)

    def chunk_body(ci, carry):
        pltpu.async_copy(table.at[idxv.at[ci]], gbuf, sem).wait()

        def row_body(r, c2):
            p0 = r * _TERMS
            wrow = wgtv[ci, pl.ds(p0, _TERMS)]
            acc = [jnp.zeros((_LANES,), jnp.float32) for _ in range(16)]
            for t in range(_TERMS):
                w = wrow[t]
                for j in range(16):
                    acc[j] = acc[j] + w * gbuf[p0 + t, pl.ds(j * _LANES, _LANES)]
            for j in range(16):
                outv[r, pl.ds(j * _LANES, _LANES)] = acc[j]
            return c2

        lax.fori_loop(0, _CH, row_body, 0)
        pltpu.sync_copy(outv, out.at[pl.ds(wid * rows_per_tile + ci * _CH, _CH)])
        return carry

    lax.fori_loop(0, nch, chunk_body, 0)


def kernel(feat0, feat1, feat2, feat3, rois):
    feats = (feat0, feat1, feat2, feat3)
    C = feat0.shape[1]
    K = rois.shape[0]
    parts = []
    bases = []
    nrows_tab = 0
    for f in feats:
        bases.append(nrows_tab)
        nrows_tab += f.shape[0] * f.shape[2] * f.shape[3]
        parts.append(jnp.transpose(f, (0, 2, 3, 1)).reshape(-1, C))
    table = jnp.concatenate(parts, axis=0)

    idx, wgt = _indices_weights(rois, bases)
    nrows = K * _OUT * _OUT
    npad = -(-nrows // (_NW * _CH)) * (_NW * _CH)
    idx = jnp.pad(idx, ((0, npad - nrows), (0, 0)))
    wgt = jnp.pad(wgt, ((0, npad - nrows), (0, 0)))
    nch = npad // (_NW * _CH)
    idxt = idx.reshape(_NW, nch, _PAIRS)
    wgtt = wgt.reshape(_NW, nch, _PAIRS).astype(jnp.float32)

    mesh = plsc.VectorSubcoreMesh(core_axis_name="c", subcore_axis_name="s")
    run = functools.partial(
        pl.kernel,
        mesh=mesh,
        out_type=jax.ShapeDtypeStruct((npad, C), jnp.float32),
        scratch_types=[
            pltpu.VMEM((nch, _PAIRS), jnp.int32),
            pltpu.VMEM((nch, _PAIRS), jnp.float32),
            pltpu.VMEM((_PAIRS, C), jnp.float32),
            pltpu.VMEM((_CH, C), jnp.float32),
            pltpu.SemaphoreType.DMA,
        ],
    )(_sc_body)
    out = run(table, idxt, wgtt)
    out = out[:nrows].reshape(K, _OUT, _OUT, C)
    return jnp.transpose(out, (0, 3, 1, 2))


# speedup vs baseline: 26.4691x; 1.0124x over previous
import functools

import jax
import jax.numpy as jnp
from jax import lax
from jax.experimental import pallas as pl
from jax.experimental.pallas import tpu as pltpu
from jax.experimental.pallas import tpu_sc as plsc

_OUT = 7
_SN = 2
_S14 = _OUT * _SN
_FINEST = 56.0
_SIZES = (256, 128, 64, 32)
_INV_STRIDES = (0.25, 0.125, 0.0625, 0.03125)
_NC = 2
_NS = 16
_NW = _NC * _NS
_CH = 8
_TERMS = 16
_PAIRS = _CH * _TERMS
_LANES = 16


def _axis_terms(lo, hi, size_i, size_f):
    K = lo.shape[0]
    roi = jnp.maximum(hi - lo, 1.0)
    binsz = roi / _OUT
    g = (jnp.arange(_S14, dtype=jnp.float32) + 0.5) / _SN
    coord = lo[:, None] + g[None, :] * binsz[:, None]
    limf = size_f[:, None]
    valid = jnp.logical_and(coord >= -1.0, coord <= limf)
    c = jnp.maximum(coord, 0.0)
    low0 = jnp.floor(c).astype(jnp.int32)
    cond = low0 >= (size_i[:, None] - 1)
    low = jnp.where(cond, size_i[:, None] - 1, low0)
    high = jnp.where(cond, size_i[:, None] - 1, low0 + 1)
    c = jnp.where(cond, limf - 1.0, c)
    l = c - low.astype(jnp.float32)
    wl = jnp.where(valid, 1.0 - l, 0.0)
    wh = jnp.where(valid, l, 0.0)
    pos = jnp.stack([low, high], axis=2).reshape(K, 2 * _S14)
    wt = jnp.stack([wl, wh], axis=2).reshape(K, 2 * _S14)
    return pos, wt


def _indices_weights(rois, bases):
    K = rois.shape[0]
    b = rois[:, 0].astype(jnp.int32)
    x1, y1, x2, y2 = rois[:, 1], rois[:, 2], rois[:, 3], rois[:, 4]
    scale = jnp.sqrt((x2 - x1 + 1.0) * (y2 - y1 + 1.0))
    lvl = jnp.clip(jnp.floor(jnp.log2(scale / _FINEST + 1e-6)), 0, 3).astype(jnp.int32)
    size = jnp.asarray(_SIZES, jnp.int32)[lvl]
    inv = jnp.asarray(_INV_STRIDES, jnp.float32)[lvl]
    base = jnp.asarray(bases, jnp.int32)[lvl] + b * size * size
    limf = size.astype(jnp.float32)
    ypos, yw = _axis_terms(y1 * inv, y2 * inv, size, limf)
    xpos, xw = _axis_terms(x1 * inv, x2 * inv, size, limf)
    idx = (base[:, None, None, None, None]
           + ypos.reshape(K, _OUT, 1, 4, 1) * size[:, None, None, None, None]
           + xpos.reshape(K, 1, _OUT, 1, 4))
    w = yw.reshape(K, _OUT, 1, 4, 1) * xw.reshape(K, 1, _OUT, 1, 4) * (1.0 / (_SN * _SN))
    return idx.reshape(K * _OUT * _OUT, _TERMS), w.reshape(K * _OUT * _OUT, _TERMS)


def _sc_body(table, idxt, wgtt, out, idxv, wgtv, gbuf, outv,
             semg0, semg1, semo0, semo1):
    wid = lax.axis_index("s") * _NC + lax.axis_index("c")
    nch = idxt.shape[1] - 2
    rows_per_tile = nch * _CH
    semg = (semg0, semg1)
    semo = (semo0, semo1)
    pltpu.sync_copy(idxt.at[wid], idxv)
    pltpu.sync_copy(wgtt.at[wid], wgtv)
    pltpu.async_copy(table.at[idxv.at[0]], gbuf.at[0], semg0)
    pltpu.async_copy(table.at[idxv.at[1]], gbuf.at[1], semg1)

    def compute_chunk(b, ci):
        def row_body(r, c2):
            p0 = r * _TERMS
            wrow = wgtv[ci, pl.ds(p0, _TERMS)]
            acc = [jnp.zeros((_LANES,), jnp.float32) for _ in range(16)]
            for t in range(_TERMS):
                w = wrow[t]
                for j in range(16):
                    acc[j] = acc[j] + w * gbuf[b, p0 + t, pl.ds(j * _LANES, _LANES)]
            for j in range(16):
                outv[b, r, pl.ds(j * _LANES, _LANES)] = acc[j]
            return c2

        lax.fori_loop(0, _CH, row_body, 0)

    def pair_body(h, carry):
        for b in range(2):
            ci = 2 * h + b
            pltpu.make_async_copy(table.at[idxv.at[ci]], gbuf.at[b], semg[b]).wait()

            @pl.when(h > 0)
            def _():
                pltpu.make_async_copy(
                    outv.at[b], out.at[pl.ds(0, _CH)], semo[b]).wait()

            compute_chunk(b, ci)
            pltpu.async_copy(
                outv.at[b],
                out.at[pl.ds(wid * rows_per_tile + ci * _CH, _CH)], semo[b])
            pltpu.async_copy(table.at[idxv.at[ci + 2]], gbuf.at[b], semg[b])
        return carry

    lax.fori_loop(0, nch // 2, pair_body, 0)
    pltpu.make_async_copy(table.at[idxv.at[nch]], gbuf.at[0], semg0).wait()
    pltpu.make_async_copy(table.at[idxv.at[nch + 1]], gbuf.at[1], semg1).wait()
    pltpu.make_async_copy(outv.at[0], out.at[pl.ds(0, _CH)], semo0).wait()
    pltpu.make_async_copy(outv.at[1], out.at[pl.ds(0, _CH)], semo1).wait()


def kernel(feat0, feat1, feat2, feat3, rois):
    feats = (feat0, feat1, feat2, feat3)
    C = feat0.shape[1]
    K = rois.shape[0]
    parts = []
    bases = []
    nrows_tab = 0
    for f in feats:
        bases.append(nrows_tab)
        nrows_tab += f.shape[0] * f.shape[2] * f.shape[3]
        parts.append(jnp.transpose(f, (0, 2, 3, 1)).reshape(-1, C))
    table = jnp.concatenate(parts, axis=0)

    idx, wgt = _indices_weights(rois, bases)
    nrows = K * _OUT * _OUT
    npad = -(-nrows // (_NW * _CH)) * (_NW * _CH)
    idx = jnp.pad(idx, ((0, npad - nrows), (0, 0)))
    wgt = jnp.pad(wgt, ((0, npad - nrows), (0, 0)))
    nch = npad // (_NW * _CH)
    idxt = jnp.pad(idx.reshape(_NW, nch, _PAIRS), ((0, 0), (0, 2), (0, 0)))
    wgtt = jnp.pad(wgt.reshape(_NW, nch, _PAIRS).astype(jnp.float32),
                   ((0, 0), (0, 2), (0, 0)))

    mesh = plsc.VectorSubcoreMesh(core_axis_name="c", subcore_axis_name="s")
    run = functools.partial(
        pl.kernel,
        mesh=mesh,
        out_type=jax.ShapeDtypeStruct((npad, C), jnp.float32),
        scratch_types=[
            pltpu.VMEM((nch + 2, _PAIRS), jnp.int32),
            pltpu.VMEM((nch + 2, _PAIRS), jnp.float32),
            pltpu.VMEM((2, _PAIRS, C), jnp.float32),
            pltpu.VMEM((2, _CH, C), jnp.float32),
            pltpu.SemaphoreType.DMA,
            pltpu.SemaphoreType.DMA,
            pltpu.SemaphoreType.DMA,
            pltpu.SemaphoreType.DMA,
        ],
    )(_sc_body)
    out = run(table, idxt, wgtt)
    out = out[:nrows].reshape(K, _OUT, _OUT, C)
    return jnp.transpose(out, (0, 3, 1, 2))


# speedup vs baseline: 27.0145x; 1.0206x over previous
import functools

import jax
import jax.numpy as jnp
from jax import lax
from jax.experimental import pallas as pl
from jax.experimental.pallas import tpu as pltpu
from jax.experimental.pallas import tpu_sc as plsc

_OUT = 7
_SN = 2
_S14 = _OUT * _SN
_FINEST = 56.0
_SIZES = (256, 128, 64, 32)
_INV_STRIDES = (0.25, 0.125, 0.0625, 0.03125)
_NC = 2
_NS = 16
_NW = _NC * _NS
_CH = 8
_TERMS = 16
_PAIRS = _CH * _TERMS
_LANES = 16


def _axis_terms(lo, hi, size_i, size_f):
    K = lo.shape[0]
    roi = jnp.maximum(hi - lo, 1.0)
    binsz = roi / _OUT
    g = (jnp.arange(_S14, dtype=jnp.float32) + 0.5) / _SN
    coord = lo[:, None] + g[None, :] * binsz[:, None]
    limf = size_f[:, None]
    valid = jnp.logical_and(coord >= -1.0, coord <= limf)
    c = jnp.maximum(coord, 0.0)
    low0 = jnp.floor(c).astype(jnp.int32)
    cond = low0 >= (size_i[:, None] - 1)
    low = jnp.where(cond, size_i[:, None] - 1, low0)
    high = jnp.where(cond, size_i[:, None] - 1, low0 + 1)
    c = jnp.where(cond, limf - 1.0, c)
    l = c - low.astype(jnp.float32)
    wl = jnp.where(valid, 1.0 - l, 0.0)
    wh = jnp.where(valid, l, 0.0)
    pos = jnp.stack([low, high], axis=2).reshape(K, 2 * _S14)
    wt = jnp.stack([wl, wh], axis=2).reshape(K, 2 * _S14)
    return pos, wt


def _indices_weights(rois, bases):
    K = rois.shape[0]
    b = rois[:, 0].astype(jnp.int32)
    x1, y1, x2, y2 = rois[:, 1], rois[:, 2], rois[:, 3], rois[:, 4]
    scale = jnp.sqrt((x2 - x1 + 1.0) * (y2 - y1 + 1.0))
    lvl = jnp.clip(jnp.floor(jnp.log2(scale / _FINEST + 1e-6)), 0, 3).astype(jnp.int32)
    size = jnp.asarray(_SIZES, jnp.int32)[lvl]
    inv = jnp.asarray(_INV_STRIDES, jnp.float32)[lvl]
    base = jnp.asarray(bases, jnp.int32)[lvl] + b * size * size
    limf = size.astype(jnp.float32)
    ypos, yw = _axis_terms(y1 * inv, y2 * inv, size, limf)
    xpos, xw = _axis_terms(x1 * inv, x2 * inv, size, limf)
    idx = (base[:, None, None, None, None]
           + ypos.reshape(K, _OUT, 1, 4, 1) * size[:, None, None, None, None]
           + xpos.reshape(K, 1, _OUT, 1, 4))
    w = yw.reshape(K, _OUT, 1, 4, 1) * xw.reshape(K, 1, _OUT, 1, 4) * (1.0 / (_SN * _SN))
    return idx.reshape(K * _OUT * _OUT, _TERMS), w.reshape(K * _OUT * _OUT, _TERMS)


def _sc_body(table, idxt, wgtt, out, idxv, wgtv, gbuf, outv,
             semg0, semg1, semo0, semo1):
    wid = lax.axis_index("s") * _NC + lax.axis_index("c")
    nch = idxt.shape[1] - 2
    rows_per_tile = nch * _CH
    semg = (semg0, semg1)
    semo = (semo0, semo1)
    pltpu.sync_copy(idxt.at[wid], idxv)
    pltpu.sync_copy(wgtt.at[wid], wgtv)
    pltpu.async_copy(table.at[idxv.at[0]], gbuf.at[0], semg0)
    pltpu.async_copy(table.at[idxv.at[1]], gbuf.at[1], semg1)

    def compute_chunk(b, ci):
        def row_body(r, c2):
            p0 = r * _TERMS
            wrow = wgtv[ci, pl.ds(p0, _TERMS)]
            dnums = lax.GatherDimensionNumbers(
                offset_dims=(), collapsed_slice_dims=(0,), start_index_map=(0,))
            wb = [lax.gather(wrow, jnp.full((_LANES, 1), t, jnp.int32),
                             dimension_numbers=dnums, slice_sizes=(1,),
                             mode=lax.GatherScatterMode.PROMISE_IN_BOUNDS)
                  for t in range(_TERMS)]
            for half in range(2):
                j0 = half * 8
                acc = [wb[0] * gbuf[b, p0, pl.ds((j0 + j) * _LANES, _LANES)]
                       for j in range(8)]
                for t in range(1, _TERMS):
                    for j in range(8):
                        acc[j] = acc[j] + wb[t] * gbuf[
                            b, p0 + t, pl.ds((j0 + j) * _LANES, _LANES)]
                for j in range(8):
                    outv[b, r, pl.ds((j0 + j) * _LANES, _LANES)] = acc[j]
            return c2

        lax.fori_loop(0, _CH, row_body, 0)

    def pair_body(h, carry):
        for b in range(2):
            ci = 2 * h + b
            pltpu.make_async_copy(table.at[idxv.at[ci]], gbuf.at[b], semg[b]).wait()

            @pl.when(h > 0)
            def _():
                pltpu.make_async_copy(
                    outv.at[b], out.at[pl.ds(0, _CH)], semo[b]).wait()

            compute_chunk(b, ci)
            pltpu.async_copy(
                outv.at[b],
                out.at[pl.ds(wid * rows_per_tile + ci * _CH, _CH)], semo[b])
            pltpu.async_copy(table.at[idxv.at[ci + 2]], gbuf.at[b], semg[b])
        return carry

    lax.fori_loop(0, nch // 2, pair_body, 0)
    pltpu.make_async_copy(table.at[idxv.at[nch]], gbuf.at[0], semg0).wait()
    pltpu.make_async_copy(table.at[idxv.at[nch + 1]], gbuf.at[1], semg1).wait()
    pltpu.make_async_copy(outv.at[0], out.at[pl.ds(0, _CH)], semo0).wait()
    pltpu.make_async_copy(outv.at[1], out.at[pl.ds(0, _CH)], semo1).wait()


def kernel(feat0, feat1, feat2, feat3, rois):
    feats = (feat0, feat1, feat2, feat3)
    C = feat0.shape[1]
    K = rois.shape[0]
    parts = []
    bases = []
    nrows_tab = 0
    for f in feats:
        bases.append(nrows_tab)
        nrows_tab += f.shape[0] * f.shape[2] * f.shape[3]
        parts.append(jnp.transpose(f, (0, 2, 3, 1)).reshape(-1, C))
    table = jnp.concatenate(parts, axis=0)

    idx, wgt = _indices_weights(rois, bases)
    nrows = K * _OUT * _OUT
    npad = -(-nrows // (_NW * _CH)) * (_NW * _CH)
    idx = jnp.pad(idx, ((0, npad - nrows), (0, 0)))
    wgt = jnp.pad(wgt, ((0, npad - nrows), (0, 0)))
    nch = npad // (_NW * _CH)
    idxt = jnp.pad(idx.reshape(_NW, nch, _PAIRS), ((0, 0), (0, 2), (0, 0)))
    wgtt = jnp.pad(wgt.reshape(_NW, nch, _PAIRS).astype(jnp.float32),
                   ((0, 0), (0, 2), (0, 0)))

    mesh = plsc.VectorSubcoreMesh(core_axis_name="c", subcore_axis_name="s")
    run = functools.partial(
        pl.kernel,
        mesh=mesh,
        out_type=jax.ShapeDtypeStruct((npad, C), jnp.float32),
        scratch_types=[
            pltpu.VMEM((nch + 2, _PAIRS), jnp.int32),
            pltpu.VMEM((nch + 2, _PAIRS), jnp.float32),
            pltpu.VMEM((2, _PAIRS, C), jnp.float32),
            pltpu.VMEM((2, _CH, C), jnp.float32),
            pltpu.SemaphoreType.DMA,
            pltpu.SemaphoreType.DMA,
            pltpu.SemaphoreType.DMA,
            pltpu.SemaphoreType.DMA,
        ],
    )(_sc_body)
    out = run(table, idxt, wgtt)
    out = out[:nrows].reshape(K, _OUT, _OUT, C)
    return jnp.transpose(out, (0, 3, 1, 2))
